# Initial kernel scaffold; baseline (speedup 1.0000x reference)
import jax
import jax.numpy as jnp
from jax.experimental import pallas as pl


def kernel(x, gate_w, gate_b, w1, b1, w2, b2):
    raise NotImplementedError("write your pallas kernel here")


# speedup vs baseline: 3.5047x; 3.5047x over previous
import functools

import jax
import jax.numpy as jnp
from jax.experimental import pallas as pl
from jax.experimental.pallas import tpu as pltpu

E = 8
K = 2
C = 768
H = 3072
T = 2048
BLK = 256
NBLK = (T * K + E * (BLK - 1) + BLK - 1) // BLK
P = NBLK * BLK


def _router_body(zt_ref, e0_ref, e1_ref, v0_ref, v1_ref, loss_ref):
    zt = zt_ref[...]
    m = jnp.max(zt, axis=0, keepdims=True)
    ez = jnp.exp(zt - m)
    gates = ez / jnp.sum(ez, axis=0, keepdims=True)

    eidx = jax.lax.broadcasted_iota(jnp.int32, (E, T), 0)
    e0 = jnp.argmax(zt, axis=0)[None, :]
    is0 = eidx == e0
    v0_ref[...] = jnp.sum(jnp.where(is0, gates, 0.0), axis=0, keepdims=True)
    z1 = jnp.where(is0, -jnp.inf, zt)
    e1 = jnp.argmax(z1, axis=0)[None, :]
    v1_ref[...] = jnp.sum(jnp.where(eidx == e1, gates, 0.0), axis=0, keepdims=True)
    e0_ref[...] = e0
    e1_ref[...] = e1

    mg = jnp.sum(gates, axis=1, keepdims=True) / T
    loss_ref[...] = jnp.sum(mg * jnp.log(mg + 1e-8), axis=0, keepdims=True)


def _ffn_body(blk_e_ref, tp_ref, xbf_ref, ptok_ref, w1_ref, b1_ref, w2_ref,
              b2_ref, y_ref):
    i = pl.program_id(0)

    @pl.when(i * BLK < tp_ref[0])
    def _compute():
        tokb = ptok_ref[...]
        tid = jax.lax.broadcasted_iota(jnp.int32, (BLK, T), 1)
        onehot = jnp.where(tokb == tid, 1.0, 0.0).astype(jnp.bfloat16)
        xb = jax.lax.dot_general(
            onehot, xbf_ref[...], (((1,), (0,)), ((), ())),
            preferred_element_type=jnp.float32).astype(jnp.bfloat16)
        w1 = w1_ref[0].astype(jnp.bfloat16)
        h = jax.lax.dot_general(
            xb, w1, (((1,), (1,)), ((), ())),
            preferred_element_type=jnp.float32) + b1_ref[0]
        h = (h * 0.5 * (1.0 + jax.lax.erf(h * (2.0 ** -0.5)))).astype(
            jnp.bfloat16)
        w2 = w2_ref[0].astype(jnp.bfloat16)
        y = jax.lax.dot_general(
            h, w2, (((1,), (1,)), ((), ())),
            preferred_element_type=jnp.float32) + b2_ref[0]
        y_ref[...] = y.astype(jnp.bfloat16)

    @pl.when(i * BLK >= tp_ref[0])
    def _zero():
        y_ref[...] = jnp.zeros_like(y_ref)


def _combine_body(y_ref, d0_ref, d1_ref, v0_ref, v1_ref, out_ref):
    pid = jax.lax.broadcasted_iota(jnp.int32, (BLK, P), 1)
    cw = (jnp.where(d0_ref[...] == pid, v0_ref[...], 0.0) +
          jnp.where(d1_ref[...] == pid, v1_ref[...], 0.0)).astype(jnp.bfloat16)
    out_ref[...] = jax.lax.dot_general(
        cw, y_ref[...], (((1,), (0,)), ((), ())),
        preferred_element_type=jnp.float32)


def kernel(x, gate_w, gate_b, w1, b1, w2, b2):
    xf = x.reshape(-1, C)

    logits = xf @ gate_w.T + gate_b
    u = jax.random.uniform(jax.random.key(42), logits.shape,
                           minval=1e-9, maxval=1.0, dtype=jnp.float32)
    z = logits - jnp.log(-jnp.log(u))

    e0, e1, v0, v1, lossv = pl.pallas_call(
        _router_body,
        grid=(1,),
        in_specs=[pl.BlockSpec((E, T), lambda i: (0, 0))],
        out_specs=[
            pl.BlockSpec((1, T), lambda i: (0, 0)),
            pl.BlockSpec((1, T), lambda i: (0, 0)),
            pl.BlockSpec((1, T), lambda i: (0, 0)),
            pl.BlockSpec((1, T), lambda i: (0, 0)),
            pl.BlockSpec((1, 1), lambda i: (0, 0)),
        ],
        out_shape=[
            jax.ShapeDtypeStruct((1, T), jnp.int32),
            jax.ShapeDtypeStruct((1, T), jnp.int32),
            jax.ShapeDtypeStruct((1, T), jnp.float32),
            jax.ShapeDtypeStruct((1, T), jnp.float32),
            jax.ShapeDtypeStruct((1, 1), jnp.float32),
        ],
    )(z.T)
    loss = lossv.reshape(())

    e_pair = jnp.concatenate([e0[0], e1[0]])
    v_pair = jnp.concatenate([v0[0], v1[0]])
    tok = jnp.tile(jnp.arange(T, dtype=jnp.int32), 2)
    order = jnp.argsort(e_pair)
    se = e_pair[order]
    cnt = jnp.zeros((E,), jnp.int32).at[e_pair].add(1)
    padded = ((cnt + BLK - 1) // BLK) * BLK
    seg = jnp.concatenate([jnp.zeros((1,), jnp.int32),
                           jnp.cumsum(padded)[:-1]])
    start = jnp.concatenate([jnp.zeros((1,), jnp.int32),
                             jnp.cumsum(cnt)[:-1]])
    rank = jnp.arange(T * K, dtype=jnp.int32) - start[se]
    dest_sorted = seg[se] + rank
    dest = jnp.zeros((T * K,), jnp.int32).at[order].set(dest_sorted)
    ptok = jnp.zeros((P, 1), jnp.int32).at[dest, 0].set(tok)
    total_padded = jnp.sum(padded)
    bstart = jnp.minimum(jnp.arange(NBLK, dtype=jnp.int32) * BLK,
                         total_padded - BLK)
    blk_e = (jnp.searchsorted(seg, bstart, side='right') - 1).astype(jnp.int32)

    y = pl.pallas_call(
        _ffn_body,
        grid_spec=pltpu.PrefetchScalarGridSpec(
            num_scalar_prefetch=2,
            grid=(NBLK,),
            in_specs=[
                pl.BlockSpec((T, C), lambda i, be, tp: (0, 0)),
                pl.BlockSpec((BLK, 1), lambda i, be, tp: (i, 0)),
                pl.BlockSpec((1, H, C), lambda i, be, tp: (be[i], 0, 0)),
                pl.BlockSpec((1, 1, H), lambda i, be, tp: (be[i], 0, 0)),
                pl.BlockSpec((1, C, H), lambda i, be, tp: (be[i], 0, 0)),
                pl.BlockSpec((1, 1, C), lambda i, be, tp: (be[i], 0, 0)),
            ],
            out_specs=pl.BlockSpec((BLK, C), lambda i, be, tp: (i, 0)),
        ),
        out_shape=jax.ShapeDtypeStruct((P, C), jnp.bfloat16),
        compiler_params=pltpu.CompilerParams(
            dimension_semantics=("arbitrary",)),
    )(blk_e, total_padded.reshape(1), xf.astype(jnp.bfloat16), ptok,
      w1, b1.reshape(E, 1, H), w2, b2.reshape(E, 1, C))

    d0 = dest[:T].reshape(T, 1)
    d1 = dest[T:].reshape(T, 1)
    out = pl.pallas_call(
        _combine_body,
        grid=(T // BLK,),
        in_specs=[
            pl.BlockSpec((P, C), lambda i: (0, 0)),
            pl.BlockSpec((BLK, 1), lambda i: (i, 0)),
            pl.BlockSpec((BLK, 1), lambda i: (i, 0)),
            pl.BlockSpec((BLK, 1), lambda i: (i, 0)),
            pl.BlockSpec((BLK, 1), lambda i: (i, 0)),
        ],
        out_specs=pl.BlockSpec((BLK, C), lambda i: (i, 0)),
        out_shape=jax.ShapeDtypeStruct((T, C), jnp.float32),
    )(y, d0, d1, v0.reshape(T, 1), v1.reshape(T, 1))

    return out.reshape(x.shape), loss


# speedup vs baseline: 3.7332x; 1.0652x over previous
import functools

import jax
import jax.numpy as jnp
from jax.experimental import pallas as pl
from jax.experimental.pallas import tpu as pltpu

E = 8
K = 2
C = 768
H = 3072
T = 2048
BLK = 256
NBLK = (T * K + E * (BLK - 1) + BLK - 1) // BLK
P = NBLK * BLK


def _router_body(zt_ref, e0_ref, e1_ref, v0_ref, v1_ref, loss_ref):
    zt = zt_ref[...]
    m = jnp.max(zt, axis=0, keepdims=True)
    ez = jnp.exp(zt - m)
    gates = ez / jnp.sum(ez, axis=0, keepdims=True)

    eidx = jax.lax.broadcasted_iota(jnp.int32, (E, T), 0)
    e0 = jnp.argmax(zt, axis=0)[None, :]
    is0 = eidx == e0
    v0_ref[...] = jnp.sum(jnp.where(is0, gates, 0.0), axis=0, keepdims=True)
    z1 = jnp.where(is0, -jnp.inf, zt)
    e1 = jnp.argmax(z1, axis=0)[None, :]
    v1_ref[...] = jnp.sum(jnp.where(eidx == e1, gates, 0.0), axis=0, keepdims=True)
    e0_ref[...] = e0
    e1_ref[...] = e1

    mg = jnp.sum(gates, axis=1, keepdims=True) / T
    loss_ref[...] = jnp.sum(mg * jnp.log(mg + 1e-8), axis=0, keepdims=True)


def _ffn_body(blk_e_ref, tp_ref, xbf_ref, ptok_ref, w1_ref, b1_ref, w2_ref,
              b2_ref, y_ref):
    i = pl.program_id(0)

    @pl.when(i * BLK < tp_ref[0])
    def _compute():
        tokb = ptok_ref[...]
        tid = jax.lax.broadcasted_iota(jnp.int32, (BLK, T), 1)
        onehot = jnp.where(tokb == tid, 1.0, 0.0).astype(jnp.bfloat16)
        xb = jax.lax.dot_general(
            onehot, xbf_ref[...], (((1,), (0,)), ((), ())),
            preferred_element_type=jnp.float32).astype(jnp.bfloat16)
        w1 = w1_ref[0].astype(jnp.bfloat16)
        h = jax.lax.dot_general(
            xb, w1, (((1,), (1,)), ((), ())),
            preferred_element_type=jnp.float32) + b1_ref[0]
        h = (h * 0.5 * (1.0 + jax.lax.erf(h * (2.0 ** -0.5)))).astype(
            jnp.bfloat16)
        w2 = w2_ref[0].astype(jnp.bfloat16)
        y = jax.lax.dot_general(
            h, w2, (((1,), (1,)), ((), ())),
            preferred_element_type=jnp.float32) + b2_ref[0]
        y_ref[...] = y.astype(jnp.bfloat16)

    @pl.when(i * BLK >= tp_ref[0])
    def _zero():
        y_ref[...] = jnp.zeros_like(y_ref)


def _combine_body(y_ref, d0_ref, d1_ref, v0_ref, v1_ref, out_ref):
    pid = jax.lax.broadcasted_iota(jnp.int32, (BLK, P), 1)
    cw = (jnp.where(d0_ref[...] == pid, v0_ref[...], 0.0) +
          jnp.where(d1_ref[...] == pid, v1_ref[...], 0.0)).astype(jnp.bfloat16)
    out_ref[...] = jax.lax.dot_general(
        cw, y_ref[...], (((1,), (0,)), ((), ())),
        preferred_element_type=jnp.float32)


def kernel(x, gate_w, gate_b, w1, b1, w2, b2):
    xf = x.reshape(-1, C)

    logits = xf @ gate_w.T + gate_b
    u = jax.random.uniform(jax.random.key(42), logits.shape,
                           minval=1e-9, maxval=1.0, dtype=jnp.float32)
    z = logits - jnp.log(-jnp.log(u))

    e0, e1, v0, v1, lossv = pl.pallas_call(
        _router_body,
        grid=(1,),
        in_specs=[pl.BlockSpec((E, T), lambda i: (0, 0))],
        out_specs=[
            pl.BlockSpec((1, T), lambda i: (0, 0)),
            pl.BlockSpec((1, T), lambda i: (0, 0)),
            pl.BlockSpec((1, T), lambda i: (0, 0)),
            pl.BlockSpec((1, T), lambda i: (0, 0)),
            pl.BlockSpec((1, 1), lambda i: (0, 0)),
        ],
        out_shape=[
            jax.ShapeDtypeStruct((1, T), jnp.int32),
            jax.ShapeDtypeStruct((1, T), jnp.int32),
            jax.ShapeDtypeStruct((1, T), jnp.float32),
            jax.ShapeDtypeStruct((1, T), jnp.float32),
            jax.ShapeDtypeStruct((1, 1), jnp.float32),
        ],
    )(z.T)
    loss = lossv.reshape(())

    e_pair = jnp.concatenate([e0[0], e1[0]])
    tok = jnp.tile(jnp.arange(T, dtype=jnp.int32), 2)
    onehot = (e_pair[:, None] == jnp.arange(E, dtype=jnp.int32)[None, :]
              ).astype(jnp.int32)
    csum = jnp.cumsum(onehot, axis=0)
    cnt = csum[-1]
    rank = jnp.take_along_axis(csum, e_pair[:, None], axis=1)[:, 0] - 1
    padded = ((cnt + BLK - 1) // BLK) * BLK
    seg = jnp.concatenate([jnp.zeros((1,), jnp.int32),
                           jnp.cumsum(padded)[:-1]])
    dest = seg[e_pair] + rank
    ptok = jnp.zeros((P, 1), jnp.int32).at[dest, 0].set(tok)
    total_padded = jnp.sum(padded)
    bstart = jnp.minimum(jnp.arange(NBLK, dtype=jnp.int32) * BLK,
                         total_padded - BLK)
    blk_e = (jnp.searchsorted(seg, bstart, side='right') - 1).astype(jnp.int32)

    y = pl.pallas_call(
        _ffn_body,
        grid_spec=pltpu.PrefetchScalarGridSpec(
            num_scalar_prefetch=2,
            grid=(NBLK,),
            in_specs=[
                pl.BlockSpec((T, C), lambda i, be, tp: (0, 0)),
                pl.BlockSpec((BLK, 1), lambda i, be, tp: (i, 0)),
                pl.BlockSpec((1, H, C), lambda i, be, tp: (be[i], 0, 0)),
                pl.BlockSpec((1, 1, H), lambda i, be, tp: (be[i], 0, 0)),
                pl.BlockSpec((1, C, H), lambda i, be, tp: (be[i], 0, 0)),
                pl.BlockSpec((1, 1, C), lambda i, be, tp: (be[i], 0, 0)),
            ],
            out_specs=pl.BlockSpec((BLK, C), lambda i, be, tp: (i, 0)),
        ),
        out_shape=jax.ShapeDtypeStruct((P, C), jnp.bfloat16),
        compiler_params=pltpu.CompilerParams(
            dimension_semantics=("arbitrary",)),
    )(blk_e, total_padded.reshape(1), xf.astype(jnp.bfloat16), ptok,
      w1, b1.reshape(E, 1, H), w2, b2.reshape(E, 1, C))

    d0 = dest[:T].reshape(T, 1)
    d1 = dest[T:].reshape(T, 1)
    out = pl.pallas_call(
        _combine_body,
        grid=(T // BLK,),
        in_specs=[
            pl.BlockSpec((P, C), lambda i: (0, 0)),
            pl.BlockSpec((BLK, 1), lambda i: (i, 0)),
            pl.BlockSpec((BLK, 1), lambda i: (i, 0)),
            pl.BlockSpec((BLK, 1), lambda i: (i, 0)),
            pl.BlockSpec((BLK, 1), lambda i: (i, 0)),
        ],
        out_specs=pl.BlockSpec((BLK, C), lambda i: (i, 0)),
        out_shape=jax.ShapeDtypeStruct((T, C), jnp.float32),
    )(y, d0, d1, v0.reshape(T, 1), v1.reshape(T, 1))

    return out.reshape(x.shape), loss


# speedup vs baseline: 4.9086x; 1.3149x over previous
import functools

import jax
import jax.numpy as jnp
from jax.experimental import pallas as pl
from jax.experimental.pallas import tpu as pltpu
from jax.experimental.pallas import tpu_sc as plsc

E = 8
K = 2
C = 768
H = 3072
T = 2048
BLK = 256
NBLK = (T * K + E * (BLK - 1) + BLK - 1) // BLK
P = NBLK * BLK
NC = 2
NS = 16
NW = NC * NS
TPW = T // NW


def _lane_cumsum(x):
    c = x
    sh = 1
    n = x.shape[1]
    while sh < n:
        z = jnp.zeros((x.shape[0], sh), c.dtype)
        c = c + jnp.concatenate([z, c[:, :n - sh]], axis=1)
        sh *= 2
    return c


def _router_body(zt_ref, d0_ref, d1_ref, v0_ref, v1_ref, blk_e_ref, tp_ref,
                 loss_ref):
    zt = zt_ref[...]
    m = jnp.max(zt, axis=0, keepdims=True)
    ez = jnp.exp(zt - m)
    gates = ez / jnp.sum(ez, axis=0, keepdims=True)

    eidx = jax.lax.broadcasted_iota(jnp.int32, (E, T), 0)
    e0 = jnp.argmax(zt, axis=0)[None, :]
    is0 = eidx == e0
    v0_ref[...] = jnp.sum(jnp.where(is0, gates, 0.0), axis=0, keepdims=True)
    z1 = jnp.where(is0, -jnp.inf, zt)
    e1 = jnp.argmax(z1, axis=0)[None, :]
    is1 = eidx == e1
    v1_ref[...] = jnp.sum(jnp.where(is1, gates, 0.0), axis=0, keepdims=True)

    mg = jnp.sum(gates, axis=1, keepdims=True) / T
    loss_ref[...] = jnp.sum(mg * jnp.log(mg + 1e-8), axis=0, keepdims=True)

    oh0 = is0.astype(jnp.int32)
    oh1 = is1.astype(jnp.int32)
    c0 = _lane_cumsum(oh0)
    c1 = _lane_cumsum(oh1)
    tot0 = c0[:, T - 1:T]
    cnt = tot0 + c1[:, T - 1:T]
    padded = ((cnt + BLK - 1) // BLK) * BLK
    tri = (jax.lax.broadcasted_iota(jnp.int32, (E, E), 0) >
           jax.lax.broadcasted_iota(jnp.int32, (E, E), 1)).astype(jnp.float32)
    seg = jax.lax.dot_general(
        tri, padded.astype(jnp.float32), (((1,), (0,)), ((), ())),
        preferred_element_type=jnp.float32).astype(jnp.int32)
    d0_ref[...] = jnp.sum(
        jnp.where(is0, seg + c0 - oh0, 0), axis=0, keepdims=True)
    d1_ref[...] = jnp.sum(
        jnp.where(is1, seg + tot0 + c1 - oh1, 0), axis=0, keepdims=True)

    total_padded = jnp.sum(padded)
    tp_ref[...] = total_padded[None, None]
    bstart = jax.lax.broadcasted_iota(jnp.int32, (1, NBLK), 1) * BLK
    bstart = jnp.minimum(bstart, total_padded - BLK)
    blk_e_ref[...] = jnp.sum(
        (seg <= bstart).astype(jnp.int32), axis=0, keepdims=True) - 1


def _ffn_body(blk_e_ref, tp_ref, xbf_ref, d0_ref, d1_ref, v0_ref, v1_ref,
              w1_ref, b1_ref, w2_ref, b2_ref, y_ref):
    i = pl.program_id(0)

    @pl.when(i * BLK < tp_ref[0])
    def _compute():
        rowid = i * BLK + jax.lax.broadcasted_iota(jnp.int32, (BLK, T), 0)
        m0 = d0_ref[...] == rowid
        m1 = d1_ref[...] == rowid
        onehot = jnp.where(m0 | m1, 1.0, 0.0).astype(jnp.bfloat16)
        pwcol = jnp.sum(jnp.where(m0, v0_ref[...], 0.0) +
                        jnp.where(m1, v1_ref[...], 0.0), axis=1,
                        keepdims=True)
        xb = jax.lax.dot_general(
            onehot, xbf_ref[...], (((1,), (0,)), ((), ())),
            preferred_element_type=jnp.float32).astype(jnp.bfloat16)
        w1 = w1_ref[0].astype(jnp.bfloat16)
        h = jax.lax.dot_general(
            xb, w1, (((1,), (1,)), ((), ())),
            preferred_element_type=jnp.float32) + b1_ref[0]
        h = (h * 0.5 * (1.0 + jax.lax.erf(h * (2.0 ** -0.5)))).astype(
            jnp.bfloat16)
        w2 = w2_ref[0].astype(jnp.bfloat16)
        y = jax.lax.dot_general(
            h, w2, (((1,), (1,)), ((), ())),
            preferred_element_type=jnp.float32) + b2_ref[0]
        y_ref[...] = y * pwcol

    @pl.when(i * BLK >= tp_ref[0])
    def _zero():
        y_ref[...] = jnp.zeros_like(y_ref)


def _sc_combine_body(y_hbm, d0_hbm, d1_hbm, out_hbm, idx0, idx1, rows0, rows1,
                     sem0, sem1):
    wid = jax.lax.axis_index("s") * NC + jax.lax.axis_index("c")
    base = wid * TPW
    pltpu.sync_copy(d0_hbm.at[pl.ds(base, TPW)], idx0)
    pltpu.sync_copy(d1_hbm.at[pl.ds(base, TPW)], idx1)
    cp0 = pltpu.async_copy(y_hbm.at[idx0], rows0, sem0)
    cp1 = pltpu.async_copy(y_hbm.at[idx1], rows1, sem1)
    cp0.wait()
    cp1.wait()

    def row_body(t, carry):
        def col_body(j, carry2):
            sl = pl.ds(j * 16, 16)
            rows0[t, sl] = rows0[t, sl] + rows1[t, sl]
            return carry2
        return jax.lax.fori_loop(0, C // 16, col_body, carry)

    jax.lax.fori_loop(0, TPW, row_body, 0)
    pltpu.sync_copy(rows0, out_hbm.at[pl.ds(base, TPW)])


def kernel(x, gate_w, gate_b, w1, b1, w2, b2):
    xf = x.reshape(-1, C)

    logits = xf @ gate_w.T + gate_b
    u = jax.random.uniform(jax.random.key(42), logits.shape,
                           minval=1e-9, maxval=1.0, dtype=jnp.float32)
    z = logits - jnp.log(-jnp.log(u))

    d0, d1, v0, v1, blk_e2, tp2, lossv = pl.pallas_call(
        _router_body,
        grid=(1,),
        in_specs=[pl.BlockSpec((E, T), lambda i: (0, 0))],
        out_specs=[
            pl.BlockSpec((1, T), lambda i: (0, 0)),
            pl.BlockSpec((1, T), lambda i: (0, 0)),
            pl.BlockSpec((1, T), lambda i: (0, 0)),
            pl.BlockSpec((1, T), lambda i: (0, 0)),
            pl.BlockSpec((1, NBLK), lambda i: (0, 0)),
            pl.BlockSpec((1, 1), lambda i: (0, 0)),
            pl.BlockSpec((1, 1), lambda i: (0, 0)),
        ],
        out_shape=[
            jax.ShapeDtypeStruct((1, T), jnp.int32),
            jax.ShapeDtypeStruct((1, T), jnp.int32),
            jax.ShapeDtypeStruct((1, T), jnp.float32),
            jax.ShapeDtypeStruct((1, T), jnp.float32),
            jax.ShapeDtypeStruct((1, NBLK), jnp.int32),
            jax.ShapeDtypeStruct((1, 1), jnp.int32),
            jax.ShapeDtypeStruct((1, 1), jnp.float32),
        ],
    )(z.T)
    loss = lossv.reshape(())

    y = pl.pallas_call(
        _ffn_body,
        grid_spec=pltpu.PrefetchScalarGridSpec(
            num_scalar_prefetch=2,
            grid=(NBLK,),
            in_specs=[
                pl.BlockSpec((T, C), lambda i, be, tp: (0, 0)),
                pl.BlockSpec((1, T), lambda i, be, tp: (0, 0)),
                pl.BlockSpec((1, T), lambda i, be, tp: (0, 0)),
                pl.BlockSpec((1, T), lambda i, be, tp: (0, 0)),
                pl.BlockSpec((1, T), lambda i, be, tp: (0, 0)),
                pl.BlockSpec((1, H, C), lambda i, be, tp: (be[i], 0, 0)),
                pl.BlockSpec((1, 1, H), lambda i, be, tp: (be[i], 0, 0)),
                pl.BlockSpec((1, C, H), lambda i, be, tp: (be[i], 0, 0)),
                pl.BlockSpec((1, 1, C), lambda i, be, tp: (be[i], 0, 0)),
            ],
            out_specs=pl.BlockSpec((BLK, C), lambda i, be, tp: (i, 0)),
        ),
        out_shape=jax.ShapeDtypeStruct((P, C), jnp.float32),
        compiler_params=pltpu.CompilerParams(
            dimension_semantics=("arbitrary",)),
    )(blk_e2.reshape(NBLK), tp2.reshape(1), xf.astype(jnp.bfloat16),
      d0, d1, v0, v1, w1, b1.reshape(E, 1, H), w2, b2.reshape(E, 1, C))

    combine = functools.partial(
        pl.kernel,
        out_type=jax.ShapeDtypeStruct((T, C), jnp.float32),
        mesh=plsc.VectorSubcoreMesh(core_axis_name="c", subcore_axis_name="s",
                                    num_cores=NC, num_subcores=NS),
        scratch_types=[
            pltpu.VMEM((TPW,), jnp.int32),
            pltpu.VMEM((TPW,), jnp.int32),
            pltpu.VMEM((TPW, C), jnp.float32),
            pltpu.VMEM((TPW, C), jnp.float32),
            pltpu.SemaphoreType.DMA,
            pltpu.SemaphoreType.DMA,
        ],
    )(_sc_combine_body)
    out = combine(y, d0.reshape(T), d1.reshape(T))

    return out.reshape(x.shape), loss


# speedup vs baseline: 4.9844x; 1.0154x over previous
import functools

import jax
import jax.numpy as jnp
from jax.experimental import pallas as pl
from jax.experimental.pallas import tpu as pltpu
from jax.experimental.pallas import tpu_sc as plsc

E = 8
K = 2
C = 768
H = 3072
T = 2048
BLK = 256
NBLK = (T * K + E * (BLK - 1) + BLK - 1) // BLK
P = NBLK * BLK
NC = 2
NS = 16
NW = NC * NS
TPW = T // NW


def _lane_cumsum(x):
    c = x
    sh = 1
    n = x.shape[1]
    while sh < n:
        z = jnp.zeros((x.shape[0], sh), c.dtype)
        c = c + jnp.concatenate([z, c[:, :n - sh]], axis=1)
        sh *= 2
    return c


def _router_body(zt_ref, d0_ref, d1_ref, v0_ref, v1_ref, blk_e_ref, tp_ref,
                 loss_ref):
    zt = zt_ref[...]
    m = jnp.max(zt, axis=0, keepdims=True)
    ez = jnp.exp(zt - m)
    gates = ez / jnp.sum(ez, axis=0, keepdims=True)

    eidx = jax.lax.broadcasted_iota(jnp.int32, (E, T), 0)
    e0 = jnp.argmax(zt, axis=0)[None, :]
    is0 = eidx == e0
    v0_ref[...] = jnp.sum(jnp.where(is0, gates, 0.0), axis=0, keepdims=True)
    z1 = jnp.where(is0, -jnp.inf, zt)
    e1 = jnp.argmax(z1, axis=0)[None, :]
    is1 = eidx == e1
    v1_ref[...] = jnp.sum(jnp.where(is1, gates, 0.0), axis=0, keepdims=True)

    mg = jnp.sum(gates, axis=1, keepdims=True) / T
    loss_ref[...] = jnp.sum(mg * jnp.log(mg + 1e-8), axis=0, keepdims=True)

    oh0 = is0.astype(jnp.int32)
    oh1 = is1.astype(jnp.int32)
    c0 = _lane_cumsum(oh0)
    c1 = _lane_cumsum(oh1)
    tot0 = c0[:, T - 1:T]
    cnt = tot0 + c1[:, T - 1:T]
    padded = ((cnt + BLK - 1) // BLK) * BLK
    tri = (jax.lax.broadcasted_iota(jnp.int32, (E, E), 0) >
           jax.lax.broadcasted_iota(jnp.int32, (E, E), 1)).astype(jnp.float32)
    seg = jax.lax.dot_general(
        tri, padded.astype(jnp.float32), (((1,), (0,)), ((), ())),
        preferred_element_type=jnp.float32).astype(jnp.int32)
    d0_ref[...] = jnp.sum(
        jnp.where(is0, seg + c0 - oh0, 0), axis=0, keepdims=True)
    d1_ref[...] = jnp.sum(
        jnp.where(is1, seg + tot0 + c1 - oh1, 0), axis=0, keepdims=True)

    total_padded = jnp.sum(padded)
    tp_ref[...] = total_padded[None, None]
    bstart = jax.lax.broadcasted_iota(jnp.int32, (1, NBLK), 1) * BLK
    bstart = jnp.minimum(bstart, total_padded - BLK)
    blk_e_ref[...] = jnp.sum(
        (seg <= bstart).astype(jnp.int32), axis=0, keepdims=True) - 1


def _ffn_body(blk_e_ref, tp_ref, xbf_ref, d0_ref, d1_ref, v0_ref, v1_ref,
              w1_ref, b1_ref, w2_ref, b2_ref, y_ref, w1c_ref, w2c_ref):
    i = pl.program_id(0)
    prev = blk_e_ref[jnp.maximum(i - 1, 0)]

    @pl.when((i == 0) | (blk_e_ref[i] != prev))
    def _recast():
        w1c_ref[...] = w1_ref[0].astype(jnp.bfloat16)
        w2c_ref[...] = w2_ref[0].astype(jnp.bfloat16)

    @pl.when(i * BLK < tp_ref[0])
    def _compute():
        rowid = i * BLK + jax.lax.broadcasted_iota(jnp.int32, (BLK, T), 0)
        m0 = d0_ref[...] == rowid
        m1 = d1_ref[...] == rowid
        onehot = jnp.where(m0 | m1, 1.0, 0.0).astype(jnp.bfloat16)
        pwcol = jnp.sum(jnp.where(m0, v0_ref[...], 0.0) +
                        jnp.where(m1, v1_ref[...], 0.0), axis=1,
                        keepdims=True)
        xb = jax.lax.dot_general(
            onehot, xbf_ref[...], (((1,), (0,)), ((), ())),
            preferred_element_type=jnp.float32).astype(jnp.bfloat16)
        h = jax.lax.dot_general(
            xb, w1c_ref[...], (((1,), (1,)), ((), ())),
            preferred_element_type=jnp.float32) + b1_ref[0]
        h = (h * 0.5 * (1.0 + jax.lax.erf(h * (2.0 ** -0.5)))).astype(
            jnp.bfloat16)
        y = jax.lax.dot_general(
            h, w2c_ref[...], (((1,), (1,)), ((), ())),
            preferred_element_type=jnp.float32) + b2_ref[0]
        y_ref[...] = y * pwcol

    @pl.when(i * BLK >= tp_ref[0])
    def _zero():
        y_ref[...] = jnp.zeros_like(y_ref)


def _sc_combine_body(y_hbm, d0_hbm, d1_hbm, out_hbm, idx0, idx1, rows0, rows1,
                     sem0, sem1):
    wid = jax.lax.axis_index("s") * NC + jax.lax.axis_index("c")
    base = wid * TPW
    pltpu.sync_copy(d0_hbm.at[pl.ds(base, TPW)], idx0)
    pltpu.sync_copy(d1_hbm.at[pl.ds(base, TPW)], idx1)
    cp0 = pltpu.async_copy(y_hbm.at[idx0], rows0, sem0)
    cp1 = pltpu.async_copy(y_hbm.at[idx1], rows1, sem1)
    cp0.wait()
    cp1.wait()

    def row_body(t, carry):
        for j in range(C // 16):
            sl = pl.ds(j * 16, 16)
            rows0[t, sl] = rows0[t, sl] + rows1[t, sl]
        return carry

    jax.lax.fori_loop(0, TPW, row_body, 0)
    pltpu.sync_copy(rows0, out_hbm.at[pl.ds(base, TPW)])


def kernel(x, gate_w, gate_b, w1, b1, w2, b2):
    xf = x.reshape(-1, C)

    logits = xf @ gate_w.T + gate_b
    u = jax.random.uniform(jax.random.key(42), logits.shape,
                           minval=1e-9, maxval=1.0, dtype=jnp.float32)
    z = logits - jnp.log(-jnp.log(u))

    d0, d1, v0, v1, blk_e2, tp2, lossv = pl.pallas_call(
        _router_body,
        grid=(1,),
        in_specs=[pl.BlockSpec((E, T), lambda i: (0, 0))],
        out_specs=[
            pl.BlockSpec((1, T), lambda i: (0, 0)),
            pl.BlockSpec((1, T), lambda i: (0, 0)),
            pl.BlockSpec((1, T), lambda i: (0, 0)),
            pl.BlockSpec((1, T), lambda i: (0, 0)),
            pl.BlockSpec((1, NBLK), lambda i: (0, 0)),
            pl.BlockSpec((1, 1), lambda i: (0, 0)),
            pl.BlockSpec((1, 1), lambda i: (0, 0)),
        ],
        out_shape=[
            jax.ShapeDtypeStruct((1, T), jnp.int32),
            jax.ShapeDtypeStruct((1, T), jnp.int32),
            jax.ShapeDtypeStruct((1, T), jnp.float32),
            jax.ShapeDtypeStruct((1, T), jnp.float32),
            jax.ShapeDtypeStruct((1, NBLK), jnp.int32),
            jax.ShapeDtypeStruct((1, 1), jnp.int32),
            jax.ShapeDtypeStruct((1, 1), jnp.float32),
        ],
    )(z.T)
    loss = lossv.reshape(())

    y = pl.pallas_call(
        _ffn_body,
        grid_spec=pltpu.PrefetchScalarGridSpec(
            num_scalar_prefetch=2,
            grid=(NBLK,),
            in_specs=[
                pl.BlockSpec((T, C), lambda i, be, tp: (0, 0)),
                pl.BlockSpec((1, T), lambda i, be, tp: (0, 0)),
                pl.BlockSpec((1, T), lambda i, be, tp: (0, 0)),
                pl.BlockSpec((1, T), lambda i, be, tp: (0, 0)),
                pl.BlockSpec((1, T), lambda i, be, tp: (0, 0)),
                pl.BlockSpec((1, H, C), lambda i, be, tp: (be[i], 0, 0)),
                pl.BlockSpec((1, 1, H), lambda i, be, tp: (be[i], 0, 0)),
                pl.BlockSpec((1, C, H), lambda i, be, tp: (be[i], 0, 0)),
                pl.BlockSpec((1, 1, C), lambda i, be, tp: (be[i], 0, 0)),
            ],
            out_specs=pl.BlockSpec((BLK, C), lambda i, be, tp: (i, 0)),
            scratch_shapes=[
                pltpu.VMEM((H, C), jnp.bfloat16),
                pltpu.VMEM((C, H), jnp.bfloat16),
            ],
        ),
        out_shape=jax.ShapeDtypeStruct((P, C), jnp.float32),
        compiler_params=pltpu.CompilerParams(
            dimension_semantics=("arbitrary",)),
    )(blk_e2.reshape(NBLK), tp2.reshape(1), xf.astype(jnp.bfloat16),
      d0, d1, v0, v1, w1, b1.reshape(E, 1, H), w2, b2.reshape(E, 1, C))

    combine = functools.partial(
        pl.kernel,
        out_type=jax.ShapeDtypeStruct((T, C), jnp.float32),
        mesh=plsc.VectorSubcoreMesh(core_axis_name="c", subcore_axis_name="s",
                                    num_cores=NC, num_subcores=NS),
        scratch_types=[
            pltpu.VMEM((TPW,), jnp.int32),
            pltpu.VMEM((TPW,), jnp.int32),
            pltpu.VMEM((TPW, C), jnp.float32),
            pltpu.VMEM((TPW, C), jnp.float32),
            pltpu.SemaphoreType.DMA,
            pltpu.SemaphoreType.DMA,
        ],
    )(_sc_combine_body)
    out = combine(y, d0.reshape(T), d1.reshape(T))

    return out.reshape(x.shape), loss
